# Initial kernel scaffold; baseline (speedup 1.0000x reference)
import jax
import jax.numpy as jnp
from jax.experimental import pallas as pl


def kernel(feats, edge_index, layer, W, b):
    raise NotImplementedError("write your pallas kernel here")


# speedup vs baseline: 4.3094x; 4.3094x over previous
import jax
import jax.numpy as jnp
from jax import lax
from jax.experimental import pallas as pl
from jax.experimental.pallas import tpu as pltpu
from jax.experimental.pallas import tpu_sc as plsc

N = 10000
E = 160000
D = 256
H = 128

NS = 16
EPT = E // NS
CH = 80
NCH = EPT // CH

NP = 2
PR = N // NP
PRP = PR + 8
TRASH = PR

Z632 = 632
Z520 = N - 15 * Z632
ZA = 320
ZB = PRP - 15 * ZA


def _mesh():
    return plsc.VectorSubcoreMesh(core_axis_name="c", subcore_axis_name="s")


def _deg_body(src3d, dst3d, zcol, ones_h, deg_out, deg_in,
              idx_v, ones_v, zv, hist_sh):
    c = lax.axis_index("c")
    s = lax.axis_index("s")

    pltpu.sync_copy(zcol, zv)

    @pl.when(s < 15)
    def _():
        pltpu.sync_copy(zv, hist_sh.at[pl.ds(s * Z632, Z632)])

    @pl.when(s == 15)
    def _():
        pltpu.sync_copy(zv.at[pl.ds(0, Z520)],
                        hist_sh.at[pl.ds(15 * Z632, Z520)])

    pltpu.sync_copy(ones_h, ones_v)

    @pl.when(c == 0)
    def _():
        pltpu.sync_copy(src3d.at[s], idx_v)

    @pl.when(c == 1)
    def _():
        pltpu.sync_copy(dst3d.at[s], idx_v)

    plsc.subcore_barrier()

    def body(j, _):
        pltpu.sync_copy(ones_v.at[pl.ds(0, CH)],
                        hist_sh.at[idx_v.at[j]], add=True)
        return 0

    lax.fori_loop(0, NCH, body, 0)
    plsc.subcore_barrier()

    def copy_out(out_ref):
        @pl.when(s < 15)
        def _():
            pltpu.sync_copy(hist_sh.at[pl.ds(s * Z632, Z632)], zv)
            pltpu.sync_copy(zv, out_ref.at[pl.ds(s * Z632, Z632)])

        @pl.when(s == 15)
        def _():
            pltpu.sync_copy(hist_sh.at[pl.ds(15 * Z632, Z520)],
                            zv.at[pl.ds(0, Z520)])
            pltpu.sync_copy(zv.at[pl.ds(0, Z520)],
                            out_ref.at[pl.ds(15 * Z632, Z520)])

    @pl.when(c == 0)
    def _():
        copy_out(deg_out)

    @pl.when(c == 1)
    def _():
        copy_out(deg_in)


def _make_deg_kernel():
    return pl.kernel(
        _deg_body,
        out_type=[jax.ShapeDtypeStruct((N,), jnp.float32),
                  jax.ShapeDtypeStruct((N,), jnp.float32)],
        mesh=_mesh(),
        scratch_types=[
            pltpu.VMEM((NCH, CH), jnp.int32),
            pltpu.VMEM((128,), jnp.float32),
            pltpu.VMEM((Z632,), jnp.float32),
            pltpu.VMEM_SHARED((N,), jnp.float32),
        ],
    )


def _agg_body(src3d, dst3d, g0, g1, zrows, agg0, agg1,
              sidx, didx, ridx, bufa, bufb, sema, semb, agg_sh):
    c = lax.axis_index("c")
    s = lax.axis_index("s")

    pltpu.sync_copy(src3d.at[s], sidx)
    pltpu.sync_copy(dst3d.at[s], didx)

    def one_pass(g_ref, out_ref, lo):
        def remap(j, _):
            for k in range(CH // 16):
                v = didx[j, pl.ds(k * 16, 16)]
                ok = (v >= lo) & (v < lo + PR)
                ridx[j, pl.ds(k * 16, 16)] = jnp.where(ok, v - lo, TRASH)
            return 0

        lax.fori_loop(0, NCH, remap, 0)

        @pl.when(s < 15)
        def _():
            pltpu.sync_copy(zrows, agg_sh.at[pl.ds(s * ZA, ZA)])

        @pl.when(s == 15)
        def _():
            pltpu.sync_copy(zrows.at[pl.ds(0, ZB)],
                            agg_sh.at[pl.ds(15 * ZA, ZB)])

        plsc.subcore_barrier()

        def gather_start(j, buf, sem):
            pltpu.async_copy(g_ref.at[sidx.at[j]], buf, sem)

        def gather_wait(j, buf, sem):
            pltpu.make_async_copy(g_ref.at[sidx.at[j]], buf, sem).wait()

        def scatter_add(j, buf):
            pltpu.sync_copy(buf, agg_sh.at[ridx.at[j]], add=True)

        gather_start(0, bufa, sema)

        def body(it, _):
            @pl.when(it % 2 == 0)
            def _():
                gather_wait(it, bufa, sema)

                @pl.when(it + 1 < NCH)
                def _():
                    gather_start(it + 1, bufb, semb)

                scatter_add(it, bufa)

            @pl.when(it % 2 == 1)
            def _():
                gather_wait(it, bufb, semb)

                @pl.when(it + 1 < NCH)
                def _():
                    gather_start(it + 1, bufa, sema)

                scatter_add(it, bufb)

            return 0

        lax.fori_loop(0, NCH, body, 0)
        plsc.subcore_barrier()

        @pl.when(s < 15)
        def _():
            pltpu.sync_copy(agg_sh.at[pl.ds(s * ZA, ZA)],
                            out_ref.at[pl.ds(lo + s * ZA, ZA)])

        @pl.when(s == 15)
        def _():
            rem = PR - 15 * ZA
            pltpu.sync_copy(agg_sh.at[pl.ds(15 * ZA, rem)],
                            out_ref.at[pl.ds(lo + 15 * ZA, rem)])

        plsc.subcore_barrier()

    def all_passes(g_ref, out_ref):
        for p in range(NP):
            one_pass(g_ref, out_ref, p * PR)

    @pl.when(c == 0)
    def _():
        all_passes(g0, agg0)

    @pl.when(c == 1)
    def _():
        all_passes(g1, agg1)


def _make_agg_kernel():
    return pl.kernel(
        _agg_body,
        out_type=[jax.ShapeDtypeStruct((N, H), jnp.float32),
                  jax.ShapeDtypeStruct((N, H), jnp.float32)],
        mesh=_mesh(),
        scratch_types=[
            pltpu.VMEM((NCH, CH), jnp.int32),
            pltpu.VMEM((NCH, CH), jnp.int32),
            pltpu.VMEM((NCH, CH), jnp.int32),
            pltpu.VMEM((CH, H), jnp.float32),
            pltpu.VMEM((CH, H), jnp.float32),
            pltpu.SemaphoreType.DMA,
            pltpu.SemaphoreType.DMA,
            pltpu.VMEM_SHARED((PRP, H), jnp.float32),
        ],
    )


def _norm_body(feats_ref, dego_ref, degi_ref, g0_ref, g1_ref, nin_ref):
    no = lax.rsqrt(jnp.maximum(dego_ref[...], 1.0))
    g = feats_ref[...] * no
    g0_ref[...] = g[:, :H]
    g1_ref[...] = g[:, H:]
    nin_ref[...] = lax.rsqrt(jnp.maximum(degi_ref[...], 1.0))


def _norm_call(feats, dego, degi):
    nb = 10
    rb = N // nb
    return pl.pallas_call(
        _norm_body,
        grid=(nb,),
        in_specs=[
            pl.BlockSpec((rb, D), lambda i: (i, 0)),
            pl.BlockSpec((rb, 1), lambda i: (i, 0)),
            pl.BlockSpec((rb, 1), lambda i: (i, 0)),
        ],
        out_specs=[
            pl.BlockSpec((rb, H), lambda i: (i, 0)),
            pl.BlockSpec((rb, H), lambda i: (i, 0)),
            pl.BlockSpec((rb, 1), lambda i: (i, 0)),
        ],
        out_shape=[jax.ShapeDtypeStruct((N, H), jnp.float32),
                   jax.ShapeDtypeStruct((N, H), jnp.float32),
                   jax.ShapeDtypeStruct((N, 1), jnp.float32)],
    )(feats, dego, degi)


def _mm_body(a0_ref, a1_ref, nin_ref, w_ref, b_ref, o_ref):
    w = w_ref[...]
    acc = jnp.dot(a0_ref[...], w[:H, :], preferred_element_type=jnp.float32)
    acc = acc + jnp.dot(a1_ref[...], w[H:, :],
                        preferred_element_type=jnp.float32)
    o_ref[...] = acc * nin_ref[...] + b_ref[...]


def _mm_call(agg0, agg1, nin, W, b2):
    nb = 10
    rb = N // nb
    return pl.pallas_call(
        _mm_body,
        grid=(nb,),
        in_specs=[
            pl.BlockSpec((rb, H), lambda i: (i, 0)),
            pl.BlockSpec((rb, H), lambda i: (i, 0)),
            pl.BlockSpec((rb, 1), lambda i: (i, 0)),
            pl.BlockSpec((D, D), lambda i: (0, 0)),
            pl.BlockSpec((1, D), lambda i: (0, 0)),
        ],
        out_specs=pl.BlockSpec((rb, D), lambda i: (i, 0)),
        out_shape=jax.ShapeDtypeStruct((N, D), jnp.float32),
    )(agg0, agg1, nin, W, b2)


@jax.jit
def _run(feats, edge_index, W, b):
    src3d = edge_index[0].astype(jnp.int32).reshape(NS, NCH, CH)
    dst3d = edge_index[1].astype(jnp.int32).reshape(NS, NCH, CH)
    zcol = jnp.zeros((Z632,), jnp.float32)
    ones_h = jnp.ones((128,), jnp.float32)
    zrows = jnp.zeros((ZA, H), jnp.float32)

    deg_out, deg_in = _make_deg_kernel()(src3d, dst3d, zcol, ones_h)
    g0, g1, nin = _norm_call(feats, deg_out.reshape(N, 1),
                             deg_in.reshape(N, 1))
    agg0, agg1 = _make_agg_kernel()(src3d, dst3d, g0, g1, zrows)
    return _mm_call(agg0, agg1, nin, W, b.reshape(1, D))


def kernel(feats, edge_index, layer, W, b):
    return _run(feats, edge_index, W, b)


# speedup vs baseline: 5.1245x; 1.1891x over previous
import jax
import jax.numpy as jnp
from jax import lax
from jax.experimental import pallas as pl
from jax.experimental.pallas import tpu as pltpu
from jax.experimental.pallas import tpu_sc as plsc

N = 10000
E = 160000
D = 256
H = 128

NS = 16
EPT = E // NS
CH = 80
NCH = EPT // CH
CC = 64
CCS = 6
CR = EPT // CC + 1

NP = 2
PR = N // NP
PRP = PR + 8
TRASH = PR

Z632 = 632
Z520 = N - 15 * Z632
ZA = 320
ZB = PRP - 15 * ZA


def _mesh():
    return plsc.VectorSubcoreMesh(core_axis_name="c", subcore_axis_name="s")


def _lane_gather(x, idx):
    return lax.gather(
        x, idx[:, None],
        lax.GatherDimensionNumbers(offset_dims=(), collapsed_slice_dims=(0,),
                                   start_index_map=(0,)),
        slice_sizes=(1,),
        mode=lax.GatherScatterMode.PROMISE_IN_BOUNDS)


def _deg_body(src3d, dst3d, zcol, ones_h, deg_out, deg_in,
              idx_v, ones_v, zv, hist_sh):
    c = lax.axis_index("c")
    s = lax.axis_index("s")

    pltpu.sync_copy(zcol, zv)

    @pl.when(s < 15)
    def _():
        pltpu.sync_copy(zv, hist_sh.at[pl.ds(s * Z632, Z632)])

    @pl.when(s == 15)
    def _():
        pltpu.sync_copy(zv.at[pl.ds(0, Z520)],
                        hist_sh.at[pl.ds(15 * Z632, Z520)])

    pltpu.sync_copy(ones_h, ones_v)

    @pl.when(c == 0)
    def _():
        pltpu.sync_copy(src3d.at[s], idx_v)

    @pl.when(c == 1)
    def _():
        pltpu.sync_copy(dst3d.at[s], idx_v)

    plsc.subcore_barrier()

    def body(j, _):
        pltpu.sync_copy(ones_v.at[pl.ds(0, CH)],
                        hist_sh.at[idx_v.at[j]], add=True)
        return 0

    lax.fori_loop(0, NCH, body, 0)
    plsc.subcore_barrier()

    def copy_out(out_ref):
        @pl.when(s < 15)
        def _():
            pltpu.sync_copy(hist_sh.at[pl.ds(s * Z632, Z632)], zv)
            pltpu.sync_copy(zv, out_ref.at[pl.ds(s * Z632, Z632)])

        @pl.when(s == 15)
        def _():
            pltpu.sync_copy(hist_sh.at[pl.ds(15 * Z632, Z520)],
                            zv.at[pl.ds(0, Z520)])
            pltpu.sync_copy(zv.at[pl.ds(0, Z520)],
                            out_ref.at[pl.ds(15 * Z632, Z520)])

    @pl.when(c == 0)
    def _():
        copy_out(deg_out)

    @pl.when(c == 1)
    def _():
        copy_out(deg_in)


def _make_deg_kernel():
    return pl.kernel(
        _deg_body,
        out_type=[jax.ShapeDtypeStruct((N,), jnp.float32),
                  jax.ShapeDtypeStruct((N,), jnp.float32)],
        mesh=_mesh(),
        scratch_types=[
            pltpu.VMEM((NCH, CH), jnp.int32),
            pltpu.VMEM((128,), jnp.float32),
            pltpu.VMEM((Z632,), jnp.float32),
            pltpu.VMEM_SHARED((N,), jnp.float32),
        ],
    )


def _agg_body(src3d, dst3d, g0, g1, zrows, trash2d, agg0, agg1,
              sidx, didx, csrc, cdst, bufa, bufb, sema, semb, agg_sh):
    c = lax.axis_index("c")
    s = lax.axis_index("s")

    pltpu.sync_copy(src3d.at[s], sidx)
    pltpu.sync_copy(dst3d.at[s], didx)

    for p in range(NP):
        lo = p * PR

        pltpu.sync_copy(trash2d, cdst)
        pltpu.sync_copy(trash2d, csrc)

        lanes = lax.iota(jnp.int32, 16)

        def comp(j, cnt):
            for k in range(CH // 16):
                d = didx[j, pl.ds(k * 16, 16)]
                v = sidx[j, pl.ds(k * 16, 16)]
                ok = (d >= lo) & (d < lo + PR)
                x = ok.astype(jnp.int32)
                for sh in (1, 2, 4, 8):
                    shifted = _lane_gather(x, jnp.maximum(lanes - sh, 0))
                    x = x + jnp.where(lanes >= sh, shifted, 0)
                pos = cnt + x - 1
                row = lax.shift_right_logical(pos, CCS)
                col = lax.bitwise_and(pos, CC - 1)
                plsc.store_scatter(cdst, [row, col], d - lo, mask=ok)
                plsc.store_scatter(csrc, [row, col], v, mask=ok)
                cnt = cnt + plsc.all_reduce_population_count(ok)
            return cnt

        cnt = lax.fori_loop(0, NCH, comp, jnp.zeros((16,), jnp.int32))
        trips = lax.shift_right_logical(jnp.max(cnt) + CC - 1, CCS)

        @pl.when(s < 15)
        def _():
            pltpu.sync_copy(zrows, agg_sh.at[pl.ds(s * ZA, ZA)])

        @pl.when(s == 15)
        def _():
            pltpu.sync_copy(zrows.at[pl.ds(0, ZB)],
                            agg_sh.at[pl.ds(15 * ZA, ZB)])

        plsc.subcore_barrier()

        def stream_pass(g_ref):
            def gather_start(j, buf, sem):
                pltpu.async_copy(g_ref.at[csrc.at[j]], buf, sem)

            def gather_wait(j, buf, sem):
                pltpu.make_async_copy(g_ref.at[csrc.at[j]], buf, sem).wait()

            def scatter_add(j, buf):
                pltpu.sync_copy(buf, agg_sh.at[cdst.at[j]], add=True)

            @pl.when(trips > 0)
            def _():
                gather_start(0, bufa, sema)

            def body(it, _):
                @pl.when(it % 2 == 0)
                def _():
                    gather_wait(it, bufa, sema)

                    @pl.when(it + 1 < trips)
                    def _():
                        gather_start(it + 1, bufb, semb)

                    scatter_add(it, bufa)

                @pl.when(it % 2 == 1)
                def _():
                    gather_wait(it, bufb, semb)

                    @pl.when(it + 1 < trips)
                    def _():
                        gather_start(it + 1, bufa, sema)

                    scatter_add(it, bufb)

                return 0

            lax.fori_loop(0, trips, body, 0)

        @pl.when(c == 0)
        def _():
            stream_pass(g0)

        @pl.when(c == 1)
        def _():
            stream_pass(g1)

        plsc.subcore_barrier()

        def copy_out(out_ref):
            @pl.when(s < 15)
            def _():
                pltpu.sync_copy(agg_sh.at[pl.ds(s * ZA, ZA)],
                                out_ref.at[pl.ds(lo + s * ZA, ZA)])

            @pl.when(s == 15)
            def _():
                rem = PR - 15 * ZA
                pltpu.sync_copy(agg_sh.at[pl.ds(15 * ZA, rem)],
                                out_ref.at[pl.ds(lo + 15 * ZA, rem)])

        @pl.when(c == 0)
        def _():
            copy_out(agg0)

        @pl.when(c == 1)
        def _():
            copy_out(agg1)

        plsc.subcore_barrier()


def _make_agg_kernel():
    return pl.kernel(
        _agg_body,
        out_type=[jax.ShapeDtypeStruct((N, H), jnp.float32),
                  jax.ShapeDtypeStruct((N, H), jnp.float32)],
        name="agg_scatter",
        mesh=_mesh(),
        compiler_params=pltpu.CompilerParams(needs_layout_passes=False),
        scratch_types=[
            pltpu.VMEM((NCH, CH), jnp.int32),
            pltpu.VMEM((NCH, CH), jnp.int32),
            pltpu.VMEM((CR, CC), jnp.int32),
            pltpu.VMEM((CR, CC), jnp.int32),
            pltpu.VMEM((CC, H), jnp.float32),
            pltpu.VMEM((CC, H), jnp.float32),
            pltpu.SemaphoreType.DMA,
            pltpu.SemaphoreType.DMA,
            pltpu.VMEM_SHARED((PRP, H), jnp.float32),
        ],
    )


def _norm_body(feats_ref, dego_ref, degi_ref, g0_ref, g1_ref, nin_ref):
    no = lax.rsqrt(jnp.maximum(dego_ref[...], 1.0))
    g = feats_ref[...] * no
    g0_ref[...] = g[:, :H]
    g1_ref[...] = g[:, H:]
    nin_ref[...] = lax.rsqrt(jnp.maximum(degi_ref[...], 1.0))


def _norm_call(feats, dego, degi):
    nb = 10
    rb = N // nb
    return pl.pallas_call(
        _norm_body,
        grid=(nb,),
        in_specs=[
            pl.BlockSpec((rb, D), lambda i: (i, 0)),
            pl.BlockSpec((rb, 1), lambda i: (i, 0)),
            pl.BlockSpec((rb, 1), lambda i: (i, 0)),
        ],
        out_specs=[
            pl.BlockSpec((rb, H), lambda i: (i, 0)),
            pl.BlockSpec((rb, H), lambda i: (i, 0)),
            pl.BlockSpec((rb, 1), lambda i: (i, 0)),
        ],
        out_shape=[jax.ShapeDtypeStruct((N, H), jnp.float32),
                   jax.ShapeDtypeStruct((N, H), jnp.float32),
                   jax.ShapeDtypeStruct((N, 1), jnp.float32)],
    )(feats, dego, degi)


def _mm_body(a0_ref, a1_ref, nin_ref, w_ref, b_ref, o_ref):
    w = w_ref[...]
    acc = jnp.dot(a0_ref[...], w[:H, :], preferred_element_type=jnp.float32)
    acc = acc + jnp.dot(a1_ref[...], w[H:, :],
                        preferred_element_type=jnp.float32)
    o_ref[...] = acc * nin_ref[...] + b_ref[...]


def _mm_call(agg0, agg1, nin, W, b2):
    nb = 10
    rb = N // nb
    return pl.pallas_call(
        _mm_body,
        grid=(nb,),
        in_specs=[
            pl.BlockSpec((rb, H), lambda i: (i, 0)),
            pl.BlockSpec((rb, H), lambda i: (i, 0)),
            pl.BlockSpec((rb, 1), lambda i: (i, 0)),
            pl.BlockSpec((D, D), lambda i: (0, 0)),
            pl.BlockSpec((1, D), lambda i: (0, 0)),
        ],
        out_specs=pl.BlockSpec((rb, D), lambda i: (i, 0)),
        out_shape=jax.ShapeDtypeStruct((N, D), jnp.float32),
    )(agg0, agg1, nin, W, b2)


@jax.jit
def _run(feats, edge_index, W, b):
    src3d = edge_index[0].astype(jnp.int32).reshape(NS, NCH, CH)
    dst3d = edge_index[1].astype(jnp.int32).reshape(NS, NCH, CH)
    zcol = jnp.zeros((Z632,), jnp.float32)
    ones_h = jnp.ones((128,), jnp.float32)
    zrows = jnp.zeros((ZA, H), jnp.float32)
    trash2d = jnp.full((CR, CC), TRASH, jnp.int32)

    deg_out, deg_in = _make_deg_kernel()(src3d, dst3d, zcol, ones_h)
    g0, g1, nin = _norm_call(feats, deg_out.reshape(N, 1),
                             deg_in.reshape(N, 1))
    agg0, agg1 = _make_agg_kernel()(src3d, dst3d, g0, g1, zrows, trash2d)
    return _mm_call(agg0, agg1, nin, W, b.reshape(1, D))


def kernel(feats, edge_index, layer, W, b):
    return _run(feats, edge_index, W, b)


# speedup vs baseline: 5.2408x; 1.0227x over previous
import jax
import jax.numpy as jnp
from jax import lax
from jax.experimental import pallas as pl
from jax.experimental.pallas import tpu as pltpu
from jax.experimental.pallas import tpu_sc as plsc

N = 10000
E = 160000
D = 256
H = 128

NS = 16
EPT = E // NS
CH = 80
NCH = EPT // CH
CC = 128
CCS = 7
CR = EPT // CC + 1

NP = 2
PR = N // NP
PRP = PR + 8
TRASH = PR

Z632 = 632
Z520 = N - 15 * Z632
ZA = 320
ZB = PRP - 15 * ZA


def _mesh():
    return plsc.VectorSubcoreMesh(core_axis_name="c", subcore_axis_name="s")


def _lane_gather(x, idx):
    return lax.gather(
        x, idx[:, None],
        lax.GatherDimensionNumbers(offset_dims=(), collapsed_slice_dims=(0,),
                                   start_index_map=(0,)),
        slice_sizes=(1,),
        mode=lax.GatherScatterMode.PROMISE_IN_BOUNDS)


def _deg_body(src3d, dst3d, zcol, ones_h, deg_out, deg_in,
              idx_v, ones_v, zv, hist_sh):
    c = lax.axis_index("c")
    s = lax.axis_index("s")

    pltpu.sync_copy(zcol, zv)

    @pl.when(s < 15)
    def _():
        pltpu.sync_copy(zv, hist_sh.at[pl.ds(s * Z632, Z632)])

    @pl.when(s == 15)
    def _():
        pltpu.sync_copy(zv.at[pl.ds(0, Z520)],
                        hist_sh.at[pl.ds(15 * Z632, Z520)])

    pltpu.sync_copy(ones_h, ones_v)

    @pl.when(c == 0)
    def _():
        pltpu.sync_copy(src3d.at[s], idx_v)

    @pl.when(c == 1)
    def _():
        pltpu.sync_copy(dst3d.at[s], idx_v)

    plsc.subcore_barrier()

    def body(j, _):
        pltpu.sync_copy(ones_v.at[pl.ds(0, CH)],
                        hist_sh.at[idx_v.at[j]], add=True)
        return 0

    lax.fori_loop(0, NCH, body, 0)
    plsc.subcore_barrier()

    def copy_out(out_ref):
        @pl.when(s < 15)
        def _():
            pltpu.sync_copy(hist_sh.at[pl.ds(s * Z632, Z632)], zv)
            pltpu.sync_copy(zv, out_ref.at[pl.ds(s * Z632, Z632)])

        @pl.when(s == 15)
        def _():
            pltpu.sync_copy(hist_sh.at[pl.ds(15 * Z632, Z520)],
                            zv.at[pl.ds(0, Z520)])
            pltpu.sync_copy(zv.at[pl.ds(0, Z520)],
                            out_ref.at[pl.ds(15 * Z632, Z520)])

    @pl.when(c == 0)
    def _():
        copy_out(deg_out)

    @pl.when(c == 1)
    def _():
        copy_out(deg_in)


def _make_deg_kernel():
    return pl.kernel(
        _deg_body,
        out_type=[jax.ShapeDtypeStruct((N,), jnp.float32),
                  jax.ShapeDtypeStruct((N,), jnp.float32)],
        mesh=_mesh(),
        scratch_types=[
            pltpu.VMEM((NCH, CH), jnp.int32),
            pltpu.VMEM((128,), jnp.float32),
            pltpu.VMEM((Z632,), jnp.float32),
            pltpu.VMEM_SHARED((N,), jnp.float32),
        ],
    )


def _agg_body(src3d, dst3d, g0, g1, zrows, trash2d, agg0, agg1,
              sidx, didx, csrc, cdst, bufa, bufb, sema, semb, agg_sh):
    c = lax.axis_index("c")
    s = lax.axis_index("s")

    pltpu.sync_copy(src3d.at[s], sidx)
    pltpu.sync_copy(dst3d.at[s], didx)

    for p in range(NP):
        lo = p * PR

        pltpu.sync_copy(trash2d, cdst)
        pltpu.sync_copy(trash2d, csrc)

        lanes = lax.iota(jnp.int32, 16)

        def comp(j, cnt):
            for k in range(CH // 16):
                d = didx[j, pl.ds(k * 16, 16)]
                v = sidx[j, pl.ds(k * 16, 16)]
                ok = (d >= lo) & (d < lo + PR)
                pos = cnt + plsc.cumsum(ok.astype(jnp.int32)) - 1
                row = lax.shift_right_logical(pos, CCS)
                col = lax.bitwise_and(pos, CC - 1)
                plsc.store_scatter(cdst, [row, col], d - lo, mask=ok)
                plsc.store_scatter(csrc, [row, col], v, mask=ok)
                cnt = cnt + plsc.all_reduce_population_count(ok)
            return cnt

        cnt = lax.fori_loop(0, NCH, comp, jnp.zeros((16,), jnp.int32))
        trips = lax.shift_right_logical(jnp.max(cnt) + CC - 1, CCS)

        @pl.when(s < 15)
        def _():
            pltpu.sync_copy(zrows, agg_sh.at[pl.ds(s * ZA, ZA)])

        @pl.when(s == 15)
        def _():
            pltpu.sync_copy(zrows.at[pl.ds(0, ZB)],
                            agg_sh.at[pl.ds(15 * ZA, ZB)])

        plsc.subcore_barrier()

        def stream_pass(g_ref):
            def gather_start(j, buf, sem):
                pltpu.async_copy(g_ref.at[csrc.at[j]], buf, sem)

            def gather_wait(j, buf, sem):
                pltpu.make_async_copy(g_ref.at[csrc.at[j]], buf, sem).wait()

            def scatter_add(j, buf):
                pltpu.sync_copy(buf, agg_sh.at[cdst.at[j]], add=True)

            @pl.when(trips > 0)
            def _():
                gather_start(0, bufa, sema)

            def body(it, _):
                @pl.when(it % 2 == 0)
                def _():
                    gather_wait(it, bufa, sema)

                    @pl.when(it + 1 < trips)
                    def _():
                        gather_start(it + 1, bufb, semb)

                    scatter_add(it, bufa)

                @pl.when(it % 2 == 1)
                def _():
                    gather_wait(it, bufb, semb)

                    @pl.when(it + 1 < trips)
                    def _():
                        gather_start(it + 1, bufa, sema)

                    scatter_add(it, bufb)

                return 0

            lax.fori_loop(0, trips, body, 0)

        @pl.when(c == 0)
        def _():
            stream_pass(g0)

        @pl.when(c == 1)
        def _():
            stream_pass(g1)

        plsc.subcore_barrier()

        def copy_out(out_ref):
            @pl.when(s < 15)
            def _():
                pltpu.sync_copy(agg_sh.at[pl.ds(s * ZA, ZA)],
                                out_ref.at[pl.ds(lo + s * ZA, ZA)])

            @pl.when(s == 15)
            def _():
                rem = PR - 15 * ZA
                pltpu.sync_copy(agg_sh.at[pl.ds(15 * ZA, rem)],
                                out_ref.at[pl.ds(lo + 15 * ZA, rem)])

        @pl.when(c == 0)
        def _():
            copy_out(agg0)

        @pl.when(c == 1)
        def _():
            copy_out(agg1)

        plsc.subcore_barrier()


def _make_agg_kernel():
    return pl.kernel(
        _agg_body,
        out_type=[jax.ShapeDtypeStruct((N, H), jnp.float32),
                  jax.ShapeDtypeStruct((N, H), jnp.float32)],
        name="agg_scatter",
        mesh=_mesh(),
        compiler_params=pltpu.CompilerParams(needs_layout_passes=False),
        scratch_types=[
            pltpu.VMEM((NCH, CH), jnp.int32),
            pltpu.VMEM((NCH, CH), jnp.int32),
            pltpu.VMEM((CR, CC), jnp.int32),
            pltpu.VMEM((CR, CC), jnp.int32),
            pltpu.VMEM((CC, H), jnp.float32),
            pltpu.VMEM((CC, H), jnp.float32),
            pltpu.SemaphoreType.DMA,
            pltpu.SemaphoreType.DMA,
            pltpu.VMEM_SHARED((PRP, H), jnp.float32),
        ],
    )


def _norm_body(feats_ref, dego_ref, degi_ref, g0_ref, g1_ref, nin_ref):
    no = lax.rsqrt(jnp.maximum(dego_ref[...], 1.0))
    g = feats_ref[...] * no
    g0_ref[...] = g[:, :H]
    g1_ref[...] = g[:, H:]
    nin_ref[...] = lax.rsqrt(jnp.maximum(degi_ref[...], 1.0))


def _norm_call(feats, dego, degi):
    nb = 10
    rb = N // nb
    return pl.pallas_call(
        _norm_body,
        grid=(nb,),
        in_specs=[
            pl.BlockSpec((rb, D), lambda i: (i, 0)),
            pl.BlockSpec((rb, 1), lambda i: (i, 0)),
            pl.BlockSpec((rb, 1), lambda i: (i, 0)),
        ],
        out_specs=[
            pl.BlockSpec((rb, H), lambda i: (i, 0)),
            pl.BlockSpec((rb, H), lambda i: (i, 0)),
            pl.BlockSpec((rb, 1), lambda i: (i, 0)),
        ],
        out_shape=[jax.ShapeDtypeStruct((N, H), jnp.float32),
                   jax.ShapeDtypeStruct((N, H), jnp.float32),
                   jax.ShapeDtypeStruct((N, 1), jnp.float32)],
    )(feats, dego, degi)


def _mm_body(a0_ref, a1_ref, nin_ref, w_ref, b_ref, o_ref):
    w = w_ref[...]
    acc = jnp.dot(a0_ref[...], w[:H, :], preferred_element_type=jnp.float32)
    acc = acc + jnp.dot(a1_ref[...], w[H:, :],
                        preferred_element_type=jnp.float32)
    o_ref[...] = acc * nin_ref[...] + b_ref[...]


def _mm_call(agg0, agg1, nin, W, b2):
    nb = 10
    rb = N // nb
    return pl.pallas_call(
        _mm_body,
        grid=(nb,),
        in_specs=[
            pl.BlockSpec((rb, H), lambda i: (i, 0)),
            pl.BlockSpec((rb, H), lambda i: (i, 0)),
            pl.BlockSpec((rb, 1), lambda i: (i, 0)),
            pl.BlockSpec((D, D), lambda i: (0, 0)),
            pl.BlockSpec((1, D), lambda i: (0, 0)),
        ],
        out_specs=pl.BlockSpec((rb, D), lambda i: (i, 0)),
        out_shape=jax.ShapeDtypeStruct((N, D), jnp.float32),
    )(agg0, agg1, nin, W, b2)


@jax.jit
def _run(feats, edge_index, W, b):
    src3d = edge_index[0].astype(jnp.int32).reshape(NS, NCH, CH)
    dst3d = edge_index[1].astype(jnp.int32).reshape(NS, NCH, CH)
    zcol = jnp.zeros((Z632,), jnp.float32)
    ones_h = jnp.ones((128,), jnp.float32)
    zrows = jnp.zeros((ZA, H), jnp.float32)
    trash2d = jnp.full((CR, CC), TRASH, jnp.int32)

    deg_out, deg_in = _make_deg_kernel()(src3d, dst3d, zcol, ones_h)
    g0, g1, nin = _norm_call(feats, deg_out.reshape(N, 1),
                             deg_in.reshape(N, 1))
    agg0, agg1 = _make_agg_kernel()(src3d, dst3d, g0, g1, zrows, trash2d)
    return _mm_call(agg0, agg1, nin, W, b.reshape(1, D))


def kernel(feats, edge_index, layer, W, b):
    return _run(feats, edge_index, W, b)


# speedup vs baseline: 5.4469x; 1.0393x over previous
import jax
import jax.numpy as jnp
from jax import lax
from jax.experimental import pallas as pl
from jax.experimental.pallas import tpu as pltpu
from jax.experimental.pallas import tpu_sc as plsc

N = 10000
E = 160000
D = 256
H = 128

NS = 16
EPT = E // NS
CH = 80
NCH = EPT // CH
CC = 128
CCS = 7
CR = EPT // CC + 2
CAP = CR * CC

NP = 2
PR = N // NP
PRP = PR + 8
TRASH = PR

Z632 = 632
Z520 = N - 15 * Z632
ZA = 320
ZB = PRP - 15 * ZA


def _mesh():
    return plsc.VectorSubcoreMesh(core_axis_name="c", subcore_axis_name="s")


def _lane_gather(x, idx):
    return lax.gather(
        x, idx[:, None],
        lax.GatherDimensionNumbers(offset_dims=(), collapsed_slice_dims=(0,),
                                   start_index_map=(0,)),
        slice_sizes=(1,),
        mode=lax.GatherScatterMode.PROMISE_IN_BOUNDS)


def _deg_body(src3d, dst3d, zcol, ones_h, deg_out, deg_in,
              idx_v, ones_v, zv, hist_sh):
    c = lax.axis_index("c")
    s = lax.axis_index("s")

    pltpu.sync_copy(zcol, zv)

    @pl.when(s < 15)
    def _():
        pltpu.sync_copy(zv, hist_sh.at[pl.ds(s * Z632, Z632)])

    @pl.when(s == 15)
    def _():
        pltpu.sync_copy(zv.at[pl.ds(0, Z520)],
                        hist_sh.at[pl.ds(15 * Z632, Z520)])

    pltpu.sync_copy(ones_h, ones_v)

    @pl.when(c == 0)
    def _():
        pltpu.sync_copy(src3d.at[s], idx_v)

    @pl.when(c == 1)
    def _():
        pltpu.sync_copy(dst3d.at[s], idx_v)

    plsc.subcore_barrier()

    def body(j, _):
        pltpu.sync_copy(ones_v.at[pl.ds(0, CH)],
                        hist_sh.at[idx_v.at[j]], add=True)
        return 0

    lax.fori_loop(0, NCH, body, 0)
    plsc.subcore_barrier()

    def copy_out(out_ref):
        @pl.when(s < 15)
        def _():
            pltpu.sync_copy(hist_sh.at[pl.ds(s * Z632, Z632)], zv)
            pltpu.sync_copy(zv, out_ref.at[pl.ds(s * Z632, Z632)])

        @pl.when(s == 15)
        def _():
            pltpu.sync_copy(hist_sh.at[pl.ds(15 * Z632, Z520)],
                            zv.at[pl.ds(0, Z520)])
            pltpu.sync_copy(zv.at[pl.ds(0, Z520)],
                            out_ref.at[pl.ds(15 * Z632, Z520)])

    @pl.when(c == 0)
    def _():
        copy_out(deg_out)

    @pl.when(c == 1)
    def _():
        copy_out(deg_in)


def _make_deg_kernel():
    return pl.kernel(
        _deg_body,
        out_type=[jax.ShapeDtypeStruct((N,), jnp.float32),
                  jax.ShapeDtypeStruct((N,), jnp.float32)],
        mesh=_mesh(),
        scratch_types=[
            pltpu.VMEM((NCH, CH), jnp.int32),
            pltpu.VMEM((128,), jnp.float32),
            pltpu.VMEM((Z632,), jnp.float32),
            pltpu.VMEM_SHARED((N,), jnp.float32),
        ],
    )


def _agg_body(src3d, dst3d, g0, g1, zrows, trash2d, agg0, agg1,
              sidx, didx, csrc, cdst, bufa, bufb, sema, semb, agg_sh):
    c = lax.axis_index("c")
    s = lax.axis_index("s")

    pltpu.sync_copy(src3d.at[s], sidx)
    pltpu.sync_copy(dst3d.at[s], didx)

    pltpu.sync_copy(trash2d, cdst)
    pltpu.sync_copy(trash2d, csrc)

    lanes = lax.iota(jnp.int32, 16)

    def rowcol(pos):
        return [lax.shift_right_logical(pos, CCS),
                lax.bitwise_and(pos, CC - 1)]

    def comp(j, carry):
        cnt0, cnt1 = carry
        for k in range(CH // 16):
            d = didx[j, pl.ds(k * 16, 16)]
            v = sidx[j, pl.ds(k * 16, 16)]
            ok0 = d < PR
            ok1 = jnp.logical_not(ok0)
            pre0 = plsc.cumsum(ok0.astype(jnp.int32))
            pre1 = (lanes + 1) - pre0
            pos0 = cnt0 + pre0 - 1
            pos1 = CAP - (cnt1 + pre1)
            plsc.store_scatter(cdst, rowcol(pos0), d, mask=ok0)
            plsc.store_scatter(csrc, rowcol(pos0), v, mask=ok0)
            plsc.store_scatter(cdst, rowcol(pos1), d - PR, mask=ok1)
            plsc.store_scatter(csrc, rowcol(pos1), v, mask=ok1)
            pop0 = plsc.all_reduce_population_count(ok0)
            cnt0 = cnt0 + pop0
            cnt1 = cnt1 + (16 - pop0)
        return cnt0, cnt1

    cnt0, cnt1 = lax.fori_loop(
        0, NCH, comp,
        (jnp.zeros((16,), jnp.int32), jnp.zeros((16,), jnp.int32)))
    trips_by_pass = (
        lax.shift_right_logical(jnp.max(cnt0) + CC - 1, CCS),
        lax.shift_right_logical(jnp.max(cnt1) + CC - 1, CCS))

    for p in range(NP):
        lo = p * PR
        trips = trips_by_pass[p]

        if p == 0:
            def rowmap(j):
                return j
        else:
            def rowmap(j):
                return CR - 1 - j

        @pl.when(s < 15)
        def _():
            pltpu.sync_copy(zrows, agg_sh.at[pl.ds(s * ZA, ZA)])

        @pl.when(s == 15)
        def _():
            pltpu.sync_copy(zrows.at[pl.ds(0, ZB)],
                            agg_sh.at[pl.ds(15 * ZA, ZB)])

        plsc.subcore_barrier()

        def stream_pass(g_ref):
            def gather_start(j, buf, sem):
                pltpu.async_copy(g_ref.at[csrc.at[rowmap(j)]], buf, sem)

            def gather_wait(j, buf, sem):
                pltpu.make_async_copy(g_ref.at[csrc.at[rowmap(j)]],
                                      buf, sem).wait()

            def scatter_add(j, buf):
                pltpu.sync_copy(buf, agg_sh.at[cdst.at[rowmap(j)]], add=True)

            @pl.when(trips > 0)
            def _():
                gather_start(0, bufa, sema)

            def body(it, _):
                @pl.when(it % 2 == 0)
                def _():
                    gather_wait(it, bufa, sema)

                    @pl.when(it + 1 < trips)
                    def _():
                        gather_start(it + 1, bufb, semb)

                    scatter_add(it, bufa)

                @pl.when(it % 2 == 1)
                def _():
                    gather_wait(it, bufb, semb)

                    @pl.when(it + 1 < trips)
                    def _():
                        gather_start(it + 1, bufa, sema)

                    scatter_add(it, bufb)

                return 0

            lax.fori_loop(0, trips, body, 0)

        @pl.when(c == 0)
        def _():
            stream_pass(g0)

        @pl.when(c == 1)
        def _():
            stream_pass(g1)

        plsc.subcore_barrier()

        def copy_out(out_ref):
            @pl.when(s < 15)
            def _():
                pltpu.sync_copy(agg_sh.at[pl.ds(s * ZA, ZA)],
                                out_ref.at[pl.ds(lo + s * ZA, ZA)])

            @pl.when(s == 15)
            def _():
                rem = PR - 15 * ZA
                pltpu.sync_copy(agg_sh.at[pl.ds(15 * ZA, rem)],
                                out_ref.at[pl.ds(lo + 15 * ZA, rem)])

        @pl.when(c == 0)
        def _():
            copy_out(agg0)

        @pl.when(c == 1)
        def _():
            copy_out(agg1)

        plsc.subcore_barrier()


def _make_agg_kernel():
    return pl.kernel(
        _agg_body,
        out_type=[jax.ShapeDtypeStruct((N, H), jnp.float32),
                  jax.ShapeDtypeStruct((N, H), jnp.float32)],
        name="agg_scatter",
        mesh=_mesh(),
        compiler_params=pltpu.CompilerParams(needs_layout_passes=False),
        scratch_types=[
            pltpu.VMEM((NCH, CH), jnp.int32),
            pltpu.VMEM((NCH, CH), jnp.int32),
            pltpu.VMEM((CR, CC), jnp.int32),
            pltpu.VMEM((CR, CC), jnp.int32),
            pltpu.VMEM((CC, H), jnp.float32),
            pltpu.VMEM((CC, H), jnp.float32),
            pltpu.SemaphoreType.DMA,
            pltpu.SemaphoreType.DMA,
            pltpu.VMEM_SHARED((PRP, H), jnp.float32),
        ],
    )


def _norm_body(feats_ref, dego_ref, degi_ref, g0_ref, g1_ref, nin_ref):
    no = lax.rsqrt(jnp.maximum(dego_ref[...], 1.0))
    g = feats_ref[...] * no
    g0_ref[...] = g[:, :H]
    g1_ref[...] = g[:, H:]
    nin_ref[...] = lax.rsqrt(jnp.maximum(degi_ref[...], 1.0))


def _norm_call(feats, dego, degi):
    nb = 10
    rb = N // nb
    return pl.pallas_call(
        _norm_body,
        grid=(nb,),
        in_specs=[
            pl.BlockSpec((rb, D), lambda i: (i, 0)),
            pl.BlockSpec((rb, 1), lambda i: (i, 0)),
            pl.BlockSpec((rb, 1), lambda i: (i, 0)),
        ],
        out_specs=[
            pl.BlockSpec((rb, H), lambda i: (i, 0)),
            pl.BlockSpec((rb, H), lambda i: (i, 0)),
            pl.BlockSpec((rb, 1), lambda i: (i, 0)),
        ],
        out_shape=[jax.ShapeDtypeStruct((N, H), jnp.float32),
                   jax.ShapeDtypeStruct((N, H), jnp.float32),
                   jax.ShapeDtypeStruct((N, 1), jnp.float32)],
    )(feats, dego, degi)


def _mm_body(a0_ref, a1_ref, nin_ref, w_ref, b_ref, o_ref):
    w = w_ref[...]
    acc = jnp.dot(a0_ref[...], w[:H, :], preferred_element_type=jnp.float32)
    acc = acc + jnp.dot(a1_ref[...], w[H:, :],
                        preferred_element_type=jnp.float32)
    o_ref[...] = acc * nin_ref[...] + b_ref[...]


def _mm_call(agg0, agg1, nin, W, b2):
    nb = 10
    rb = N // nb
    return pl.pallas_call(
        _mm_body,
        grid=(nb,),
        in_specs=[
            pl.BlockSpec((rb, H), lambda i: (i, 0)),
            pl.BlockSpec((rb, H), lambda i: (i, 0)),
            pl.BlockSpec((rb, 1), lambda i: (i, 0)),
            pl.BlockSpec((D, D), lambda i: (0, 0)),
            pl.BlockSpec((1, D), lambda i: (0, 0)),
        ],
        out_specs=pl.BlockSpec((rb, D), lambda i: (i, 0)),
        out_shape=jax.ShapeDtypeStruct((N, D), jnp.float32),
    )(agg0, agg1, nin, W, b2)


@jax.jit
def _run(feats, edge_index, W, b):
    src3d = edge_index[0].astype(jnp.int32).reshape(NS, NCH, CH)
    dst3d = edge_index[1].astype(jnp.int32).reshape(NS, NCH, CH)
    zcol = jnp.zeros((Z632,), jnp.float32)
    ones_h = jnp.ones((128,), jnp.float32)
    zrows = jnp.zeros((ZA, H), jnp.float32)
    trash2d = jnp.full((CR, CC), TRASH, jnp.int32)

    deg_out, deg_in = _make_deg_kernel()(src3d, dst3d, zcol, ones_h)
    g0, g1, nin = _norm_call(feats, deg_out.reshape(N, 1),
                             deg_in.reshape(N, 1))
    agg0, agg1 = _make_agg_kernel()(src3d, dst3d, g0, g1, zrows, trash2d)
    return _mm_call(agg0, agg1, nin, W, b.reshape(1, D))


def kernel(feats, edge_index, layer, W, b):
    return _run(feats, edge_index, W, b)
